# baseline (speedup 1.0000x reference)
import functools

import jax
import jax.numpy as jnp
from jax import lax
from jax.experimental import pallas as pl
from jax.experimental.pallas import tpu as pltpu
from jax.experimental.pallas import tpu_sc as plsc

N = 10000
E = 160000
D = 128

NC = 2
NS = 16
LANES = 16

NW = NC * NS
CHF = 128
CHW = 40
EPS = E // NS
NCHF = EPS // CHF
TAILF = EPS - NCHF * CHF
EPW = E // NW
NCHW = EPW // CHW
NB = 3
ROUNDS = NCHF // NB
NB16 = 25
ROUNDS16 = NCHW // NB16
NPD = 10240
RPS = NPD // NS
ZR = 128

_SC_PARAMS = pltpu.CompilerParams(use_tc_tiling_on_sc=False)


@functools.cache
def _mesh():
    return plsc.VectorSubcoreMesh(core_axis_name="c", subcore_axis_name="s",
                                  num_cores=NC, num_subcores=NS)


def _zero2d(ref, rows, cols):
    cpr = cols // LANES

    def body(i, _):
        r = i // cpr
        jc = (i % cpr) * LANES
        ref[r, pl.ds(jc, LANES)] = jnp.zeros((LANES,), jnp.float32)
        return 0

    lax.fori_loop(0, rows * cpr, body, 0)


def _pipe(table, idx_s, idx_d, rows, acc, gsem, ssem, ch, nb, rounds,
          tail=0):
    dummy = table.at[pl.ds(0, ch)]

    def sl(ref, k):
        return ref.at[pl.ds(k * ch, ch)]

    for b in range(nb):
        pltpu.async_copy(table.at[sl(idx_s, b)], rows.at[0].at[b], gsem)

    def round_(j, _):
        g = lax.rem(j, 2)
        base = j * nb
        for b in range(nb):
            pltpu.make_async_copy(dummy, rows.at[g].at[b], gsem).wait()

        @pl.when(j >= 1)
        def _():
            for b in range(nb):
                pltpu.make_async_copy(dummy, rows.at[g].at[b], ssem).wait()

        @pl.when(j + 1 < rounds)
        def _():
            for b in range(nb):
                pltpu.async_copy(table.at[sl(idx_s, base + nb + b)],
                                 rows.at[1 - g].at[b], gsem)

        for b in range(nb):
            pltpu.async_copy(rows.at[g].at[b], acc.at[sl(idx_d, base + b)],
                             ssem, add=True)
        return 0

    lax.fori_loop(0, rounds, round_, 0)
    for b in range(nb):
        pltpu.make_async_copy(dummy, rows.at[0].at[b], ssem).wait()
    if tail:
        toff = rounds * nb * ch
        tbuf = rows.at[0].at[0].at[pl.ds(0, tail)]
        tdummy = table.at[pl.ds(0, tail)]
        pltpu.async_copy(table.at[idx_s.at[pl.ds(toff, tail)]], tbuf, gsem)
        pltpu.make_async_copy(tdummy, tbuf, gsem).wait()
        pltpu.async_copy(tbuf, acc.at[idx_d.at[pl.ds(toff, tail)]], ssem,
                         add=True)
        pltpu.make_async_copy(tdummy, tbuf, ssem).wait()


def _spmm_body(Fc, sup, edges, out, idx_s, idx_d, rows, zbuf, acc, gsem,
               ssem):
    c = lax.axis_index("c")
    s = lax.axis_index("s")
    _zero2d(zbuf, ZR, Fc)
    for t in range(RPS // ZR):
        pltpu.sync_copy(zbuf, acc.at[pl.ds(s * RPS + t * ZR, ZR)])
    pltpu.sync_copy(edges.at[0].at[pl.ds(s * EPS, EPS)], idx_s)
    pltpu.sync_copy(edges.at[1].at[pl.ds(s * EPS, EPS)], idx_d)
    plsc.subcore_barrier()
    _pipe(sup.at[c], idx_s, idx_d, rows, acc, gsem, ssem, CHF, NB,
          ROUNDS, TAILF)
    plsc.subcore_barrier()
    pltpu.sync_copy(acc.at[pl.ds(s * RPS, RPS)],
                    out.at[c].at[pl.ds(s * RPS, RPS)])


@functools.partial(jax.jit, static_argnums=(2,))
def _spmm_call(sup, edges, Fc):
    return pl.kernel(
        functools.partial(_spmm_body, Fc),
        out_type=jax.ShapeDtypeStruct((2, NPD, Fc), jnp.float32),
        mesh=_mesh(),
        compiler_params=_SC_PARAMS,
        scratch_types=[
            pltpu.VMEM((EPS,), jnp.int32),
            pltpu.VMEM((EPS,), jnp.int32),
            pltpu.VMEM((2, NB, CHF, Fc), jnp.float32),
            pltpu.VMEM((ZR, Fc), jnp.float32),
            pltpu.VMEM_SHARED((NPD, Fc), jnp.float32),
            pltpu.SemaphoreType.DMA,
            pltpu.SemaphoreType.DMA,
        ],
    )(sup, edges)


def _spmm16_body(sup, edges, out, idx_s, idx_d, rows, zbuf, acc, gsem,
                 ssem):
    c = lax.axis_index("c")
    s = lax.axis_index("s")
    wid = c * NS + s
    _zero2d(zbuf, ZR, 16)
    for t in range(RPS // ZR):
        pltpu.sync_copy(zbuf, acc.at[pl.ds(s * RPS + t * ZR, ZR)])
    pltpu.sync_copy(edges.at[0].at[pl.ds(wid * EPW, EPW)], idx_s)
    pltpu.sync_copy(edges.at[1].at[pl.ds(wid * EPW, EPW)], idx_d)
    plsc.subcore_barrier()
    _pipe(sup, idx_s, idx_d, rows, acc, gsem, ssem, CHW, NB16,
          ROUNDS16)
    plsc.subcore_barrier()
    pltpu.sync_copy(acc.at[pl.ds(s * RPS, RPS)],
                    out.at[c].at[pl.ds(s * RPS, RPS)])


@jax.jit
def _spmm16_call(sup, edges):
    return pl.kernel(
        _spmm16_body,
        out_type=jax.ShapeDtypeStruct((2, NPD, 16), jnp.float32),
        mesh=_mesh(),
        compiler_params=_SC_PARAMS,
        scratch_types=[
            pltpu.VMEM((EPW,), jnp.int32),
            pltpu.VMEM((EPW,), jnp.int32),
            pltpu.VMEM((2, NB16, CHW, 16), jnp.float32),
            pltpu.VMEM((ZR, 16), jnp.float32),
            pltpu.VMEM_SHARED((NPD, 16), jnp.float32),
            pltpu.SemaphoreType.DMA,
            pltpu.SemaphoreType.DMA,
        ],
    )(sup, edges)


MB = 2000
GRID = N // MB


def _dinv(degp):
    s = degp[0] + degp[1]
    return lax.rsqrt(jnp.maximum(s[:, 0:1], 1.0))


def _elu(x):
    return jnp.where(x > 0, x, jnp.exp(jnp.minimum(x, 0.0)) - 1.0)


def _mm(a, b):
    return jnp.dot(a, b, preferred_element_type=jnp.float32)


def _row_spec(fc):
    return pl.BlockSpec((MB, fc), lambda i: (i, 0))


def _split_spec(fc):
    return pl.BlockSpec((2, MB, fc), lambda i: (0, i, 0))


def _full_spec(shp):
    return pl.BlockSpec(shp, lambda i: tuple(0 for _ in shp))


def _abc_kernel(xe, xi, deg, Wg, bg, Wi, bi, Wp, bp, Ws, osea, oseb, osia,
                osib):
    di = _dinv(deg[...])
    for x, Wpre, bpre, oa, ob in ((xe, Wg, bg, osea, oseb),
                                  (xi, Wi, bi, osia, osib)):
        f = _elu(_mm(x[...], Wpre[...]) + bpre[...])
        p = _elu(_mm(f, Wp[...]) + bp[...])
        sup = _mm(p, Ws[...]) * di
        oa[0] = sup[:, 0:64]
        oa[1] = sup[:, 64:128]
        ob[0] = sup[:, 128:192]
        ob[1] = sup[:, 192:256]


@jax.jit
def _abc_call(xe, xi, deg, Wg, bg, Wi, bi, Wp, bp, Ws):
    return pl.pallas_call(
        _abc_kernel,
        grid=(GRID,),
        in_specs=[_row_spec(128), _row_spec(128), _split_spec(16),
                  _full_spec((128, 1024)), _full_spec((1, 1024)),
                  _full_spec((128, 1024)), _full_spec((1, 1024)),
                  _full_spec((1024, 512)), _full_spec((1, 512)),
                  _full_spec((512, 256))],
        out_specs=[_split_spec(64)] * 4,
        out_shape=[jax.ShapeDtypeStruct((2, NPD, 64), jnp.float32)] * 4,
    )(xe, xi, deg, Wg, bg, Wi, bi, Wp, bp, Ws)


def _e_kernel(sera, serb, sira, sirb, deg, We, Wig, oze, ozi):
    di = _dinv(deg[...])
    pa, pb = sera[...], serb[...]
    se = jax.nn.relu(jnp.concatenate([pa[0], pa[1], pb[0], pb[1]],
                                     axis=1) * di)
    sze = _mm(se, We[...]) * di
    oze[0] = sze[:, :64]
    oze[1] = sze[:, 64:]
    pa, pb = sira[...], sirb[...]
    si = jax.nn.relu(jnp.concatenate([pa[0], pa[1], pb[0], pb[1]],
                                     axis=1) * di)
    ozi[...] = _mm(si, Wig[...]) * di


@jax.jit
def _e_call(sera, serb, sira, sirb, deg, We, Wig):
    return pl.pallas_call(
        _e_kernel,
        grid=(GRID,),
        in_specs=[_split_spec(64)] * 4 + [_split_spec(16),
                  _full_spec((256, 128)), _full_spec((256, 16))],
        out_specs=[_split_spec(64), _row_spec(16)],
        out_shape=[jax.ShapeDtypeStruct((2, NPD, 64), jnp.float32),
                   jax.ShapeDtypeStruct((NPD, 16), jnp.float32)],
    )(sera, serb, sira, sirb, deg, We, Wig)


def _g_kernel(zer, zir, deg, Wv1, oh1):
    di = _dinv(deg[...])
    zp = zer[...]
    ze = jax.nn.relu(jnp.concatenate([zp[0], zp[1]], axis=1) * di)
    ip = zir[...]
    zi = jax.nn.relu((ip[0] + ip[1]) * di)
    h = jnp.concatenate([ze, zi], axis=1)
    sh1 = _mm(h, Wv1[...]) * di
    oh1[0] = sh1[:, :32]
    oh1[1] = sh1[:, 32:]


@jax.jit
def _g_call(zer, zir, deg, Wv1):
    return pl.pallas_call(
        _g_kernel,
        grid=(GRID,),
        in_specs=[_split_spec(64), _split_spec(16), _split_spec(16),
                  _full_spec((144, 64))],
        out_specs=_split_spec(32),
        out_shape=jax.ShapeDtypeStruct((2, NPD, 32), jnp.float32),
    )(zer, zir, deg, Wv1)


def _i_kernel(h1r, deg, Wmu, omu):
    di = _dinv(deg[...])
    hp = h1r[...]
    h1 = jax.nn.relu(jnp.concatenate([hp[0], hp[1]], axis=1) * di)
    omu[...] = _mm(h1, Wmu[...]) * di


@jax.jit
def _i_call(h1r, deg, Wmu):
    return pl.pallas_call(
        _i_kernel,
        grid=(GRID,),
        in_specs=[_split_spec(32), _split_spec(16), _full_spec((64, 16))],
        out_specs=_row_spec(16),
        out_shape=jax.ShapeDtypeStruct((NPD, 16), jnp.float32),
    )(h1r, deg, Wmu)


def _k_kernel(mur, zer, deg, Wf, bf, Wd, bd, orec):
    di = _dinv(deg[...])
    mp = mur[...]
    z = (mp[0] + mp[1]) * di
    zp = zer[...]
    ze = jax.nn.relu(jnp.concatenate([zp[0], zp[1]], axis=1) * di)
    fus = _elu(_mm(jnp.concatenate([z, ze], axis=1), Wf[...]) + bf[...])
    orec[...] = _mm(fus, Wd[...]) + bd[...]


@jax.jit
def _k_call(mur, zer, deg, Wf, bf, Wd, bd):
    return pl.pallas_call(
        _k_kernel,
        grid=(GRID,),
        in_specs=[_split_spec(16), _split_spec(64), _split_spec(16),
                  _full_spec((144, 128)), _full_spec((1, 128)),
                  _full_spec((128, 128)), _full_spec((1, 128))],
        out_specs=_row_spec(128),
        out_shape=jax.ShapeDtypeStruct((N, D), jnp.float32),
    )(mur, zer, deg, Wf, bf, Wd, bd)


def kernel(x_exp, x_img, edge_index, Wg, bg, Wi, bi, Wp, bp, Ws, We, Wig,
           Wv1, Wmu, Wf, bf, Wd, bd):
    degp = _spmm16_call(jnp.ones((NPD, 16), jnp.float32), edge_index)
    sup_sea, sup_seb, sup_sia, sup_sib = _abc_call(
        x_exp, x_img, degp, Wg, bg.reshape(1, -1), Wi, bi.reshape(1, -1), Wp,
        bp.reshape(1, -1), Ws)
    se_raw_a = _spmm_call(sup_sea, edge_index, 64)
    se_raw_b = _spmm_call(sup_seb, edge_index, 64)
    si_raw_a = _spmm_call(sup_sia, edge_index, 64)
    si_raw_b = _spmm_call(sup_sib, edge_index, 64)
    sup_ze, sup_zi = _e_call(se_raw_a, se_raw_b, si_raw_a, si_raw_b, degp,
                             We, Wig)
    ze_raw = _spmm_call(sup_ze, edge_index, 64)
    zi_raw = _spmm16_call(sup_zi, edge_index)
    sup_h1 = _g_call(ze_raw, zi_raw, degp, Wv1)
    h1_raw = _spmm_call(sup_h1, edge_index, 32)
    sup_mu = _i_call(h1_raw, degp, Wmu)
    mu_raw = _spmm16_call(sup_mu, edge_index)
    return _k_call(mu_raw, ze_raw, degp, Wf, bf.reshape(1, -1), Wd,
                   bd.reshape(1, -1))


# speedup vs baseline: 1.0065x; 1.0065x over previous
import functools

import jax
import jax.numpy as jnp
from jax import lax
from jax.experimental import pallas as pl
from jax.experimental.pallas import tpu as pltpu
from jax.experimental.pallas import tpu_sc as plsc

N = 10000
E = 160000
D = 128

NC = 2
NS = 16
LANES = 16

NW = NC * NS
CHF = 80
CHW = 40
EPS = E // NS
NCHF = EPS // CHF
EPW = E // NW
NCHW = EPW // CHW
NB = 5
ROUNDS = NCHF // NB
NB16 = 25
ROUNDS16 = NCHW // NB16
NPD = 10240
RPS = NPD // NS
ZR = 128

_SC_PARAMS = pltpu.CompilerParams(use_tc_tiling_on_sc=False)


@functools.cache
def _mesh():
    return plsc.VectorSubcoreMesh(core_axis_name="c", subcore_axis_name="s",
                                  num_cores=NC, num_subcores=NS)


def _zero2d(ref, rows, cols):
    cpr = cols // LANES

    def body(i, _):
        r = i // cpr
        jc = (i % cpr) * LANES
        ref[r, pl.ds(jc, LANES)] = jnp.zeros((LANES,), jnp.float32)
        return 0

    lax.fori_loop(0, rows * cpr, body, 0)


def _pipe(table, idx_s, idx_d, rows, acc, gsem, ssem, ch, nb, rounds):
    dummy = table.at[pl.ds(0, ch)]

    def sl(ref, k):
        return ref.at[pl.ds(k * ch, ch)]

    for b in range(nb):
        pltpu.async_copy(table.at[sl(idx_s, b)], rows.at[0].at[b], gsem)

    def round_(j, _):
        g = lax.rem(j, 2)
        base = j * nb
        for b in range(nb):
            pltpu.make_async_copy(dummy, rows.at[g].at[b], gsem).wait()

        @pl.when(j >= 1)
        def _():
            for b in range(nb):
                pltpu.make_async_copy(dummy, rows.at[g].at[b], ssem).wait()

        @pl.when(j + 1 < rounds)
        def _():
            for b in range(nb):
                pltpu.async_copy(table.at[sl(idx_s, base + nb + b)],
                                 rows.at[1 - g].at[b], gsem)

        for b in range(nb):
            pltpu.async_copy(rows.at[g].at[b], acc.at[sl(idx_d, base + b)],
                             ssem, add=True)
        return 0

    lax.fori_loop(0, rounds, round_, 0)
    for b in range(nb):
        pltpu.make_async_copy(dummy, rows.at[0].at[b], ssem).wait()


def _spmm_body(Fc, sup, edges, out, idx_s, idx_d, rows, zbuf, acc, gsem,
               ssem):
    c = lax.axis_index("c")
    s = lax.axis_index("s")
    _zero2d(zbuf, ZR, Fc)
    for t in range(RPS // ZR):
        pltpu.sync_copy(zbuf, acc.at[pl.ds(s * RPS + t * ZR, ZR)])
    pltpu.sync_copy(edges.at[0].at[pl.ds(s * EPS, EPS)], idx_s)
    pltpu.sync_copy(edges.at[1].at[pl.ds(s * EPS, EPS)], idx_d)
    plsc.subcore_barrier()
    _pipe(sup.at[c], idx_s, idx_d, rows, acc, gsem, ssem, CHF, NB,
          ROUNDS)
    plsc.subcore_barrier()
    pltpu.sync_copy(acc.at[pl.ds(s * RPS, RPS)],
                    out.at[c].at[pl.ds(s * RPS, RPS)])


@functools.partial(jax.jit, static_argnums=(2,))
def _spmm_call(sup, edges, Fc):
    return pl.kernel(
        functools.partial(_spmm_body, Fc),
        out_type=jax.ShapeDtypeStruct((2, NPD, Fc), jnp.float32),
        mesh=_mesh(),
        compiler_params=_SC_PARAMS,
        scratch_types=[
            pltpu.VMEM((EPS,), jnp.int32),
            pltpu.VMEM((EPS,), jnp.int32),
            pltpu.VMEM((2, NB, CHF, Fc), jnp.float32),
            pltpu.VMEM((ZR, Fc), jnp.float32),
            pltpu.VMEM_SHARED((NPD, Fc), jnp.float32),
            pltpu.SemaphoreType.DMA,
            pltpu.SemaphoreType.DMA,
        ],
    )(sup, edges)


def _spmm16_body(sup, edges, out, idx_s, idx_d, rows, zbuf, acc, gsem,
                 ssem):
    c = lax.axis_index("c")
    s = lax.axis_index("s")
    wid = c * NS + s
    _zero2d(zbuf, ZR, 16)
    for t in range(RPS // ZR):
        pltpu.sync_copy(zbuf, acc.at[pl.ds(s * RPS + t * ZR, ZR)])
    pltpu.sync_copy(edges.at[0].at[pl.ds(wid * EPW, EPW)], idx_s)
    pltpu.sync_copy(edges.at[1].at[pl.ds(wid * EPW, EPW)], idx_d)
    plsc.subcore_barrier()
    _pipe(sup, idx_s, idx_d, rows, acc, gsem, ssem, CHW, NB16,
          ROUNDS16)
    plsc.subcore_barrier()
    pltpu.sync_copy(acc.at[pl.ds(s * RPS, RPS)],
                    out.at[c].at[pl.ds(s * RPS, RPS)])


@jax.jit
def _spmm16_call(sup, edges):
    return pl.kernel(
        _spmm16_body,
        out_type=jax.ShapeDtypeStruct((2, NPD, 16), jnp.float32),
        mesh=_mesh(),
        compiler_params=_SC_PARAMS,
        scratch_types=[
            pltpu.VMEM((EPW,), jnp.int32),
            pltpu.VMEM((EPW,), jnp.int32),
            pltpu.VMEM((2, NB16, CHW, 16), jnp.float32),
            pltpu.VMEM((ZR, 16), jnp.float32),
            pltpu.VMEM_SHARED((NPD, 16), jnp.float32),
            pltpu.SemaphoreType.DMA,
            pltpu.SemaphoreType.DMA,
        ],
    )(sup, edges)


MB = 2000
GRID = N // MB


def _dinv(degp):
    s = degp[0] + degp[1]
    return lax.rsqrt(jnp.maximum(s[:, 0:1], 1.0))


def _elu(x):
    return jnp.where(x > 0, x, jnp.exp(jnp.minimum(x, 0.0)) - 1.0)


def _mm(a, b):
    return jnp.dot(a, b, preferred_element_type=jnp.float32)


def _row_spec(fc):
    return pl.BlockSpec((MB, fc), lambda i: (i, 0))


def _split_spec(fc):
    return pl.BlockSpec((2, MB, fc), lambda i: (0, i, 0))


def _full_spec(shp):
    return pl.BlockSpec(shp, lambda i: tuple(0 for _ in shp))


def _abc_kernel(xe, xi, deg, Wg, bg, Wi, bi, Wp, bp, Ws, osea, oseb, osia,
                osib):
    di = _dinv(deg[...])
    for x, Wpre, bpre, oa, ob in ((xe, Wg, bg, osea, oseb),
                                  (xi, Wi, bi, osia, osib)):
        f = _elu(_mm(x[...], Wpre[...]) + bpre[...])
        p = _elu(_mm(f, Wp[...]) + bp[...])
        sup = _mm(p, Ws[...]) * di
        oa[0] = sup[:, 0:64]
        oa[1] = sup[:, 64:128]
        ob[0] = sup[:, 128:192]
        ob[1] = sup[:, 192:256]


@jax.jit
def _abc_call(xe, xi, deg, Wg, bg, Wi, bi, Wp, bp, Ws):
    return pl.pallas_call(
        _abc_kernel,
        grid=(GRID,),
        in_specs=[_row_spec(128), _row_spec(128), _split_spec(16),
                  _full_spec((128, 1024)), _full_spec((1, 1024)),
                  _full_spec((128, 1024)), _full_spec((1, 1024)),
                  _full_spec((1024, 512)), _full_spec((1, 512)),
                  _full_spec((512, 256))],
        out_specs=[_split_spec(64)] * 4,
        out_shape=[jax.ShapeDtypeStruct((2, NPD, 64), jnp.float32)] * 4,
    )(xe, xi, deg, Wg, bg, Wi, bi, Wp, bp, Ws)


def _e_kernel(sera, serb, sira, sirb, deg, We, Wig, oze, ozi):
    di = _dinv(deg[...])
    pa, pb = sera[...], serb[...]
    se = jax.nn.relu(jnp.concatenate([pa[0], pa[1], pb[0], pb[1]],
                                     axis=1) * di)
    sze = _mm(se, We[...]) * di
    oze[0] = sze[:, :64]
    oze[1] = sze[:, 64:]
    pa, pb = sira[...], sirb[...]
    si = jax.nn.relu(jnp.concatenate([pa[0], pa[1], pb[0], pb[1]],
                                     axis=1) * di)
    ozi[...] = _mm(si, Wig[...]) * di


@jax.jit
def _e_call(sera, serb, sira, sirb, deg, We, Wig):
    return pl.pallas_call(
        _e_kernel,
        grid=(GRID,),
        in_specs=[_split_spec(64)] * 4 + [_split_spec(16),
                  _full_spec((256, 128)), _full_spec((256, 16))],
        out_specs=[_split_spec(64), _row_spec(16)],
        out_shape=[jax.ShapeDtypeStruct((2, NPD, 64), jnp.float32),
                   jax.ShapeDtypeStruct((NPD, 16), jnp.float32)],
    )(sera, serb, sira, sirb, deg, We, Wig)


def _g_kernel(zer, zir, deg, Wv1, oh1):
    di = _dinv(deg[...])
    zp = zer[...]
    ze = jax.nn.relu(jnp.concatenate([zp[0], zp[1]], axis=1) * di)
    ip = zir[...]
    zi = jax.nn.relu((ip[0] + ip[1]) * di)
    h = jnp.concatenate([ze, zi], axis=1)
    sh1 = _mm(h, Wv1[...]) * di
    oh1[0] = sh1[:, :32]
    oh1[1] = sh1[:, 32:]


@jax.jit
def _g_call(zer, zir, deg, Wv1):
    return pl.pallas_call(
        _g_kernel,
        grid=(GRID,),
        in_specs=[_split_spec(64), _split_spec(16), _split_spec(16),
                  _full_spec((144, 64))],
        out_specs=_split_spec(32),
        out_shape=jax.ShapeDtypeStruct((2, NPD, 32), jnp.float32),
    )(zer, zir, deg, Wv1)


def _i_kernel(h1r, deg, Wmu, omu):
    di = _dinv(deg[...])
    hp = h1r[...]
    h1 = jax.nn.relu(jnp.concatenate([hp[0], hp[1]], axis=1) * di)
    omu[...] = _mm(h1, Wmu[...]) * di


@jax.jit
def _i_call(h1r, deg, Wmu):
    return pl.pallas_call(
        _i_kernel,
        grid=(GRID,),
        in_specs=[_split_spec(32), _split_spec(16), _full_spec((64, 16))],
        out_specs=_row_spec(16),
        out_shape=jax.ShapeDtypeStruct((NPD, 16), jnp.float32),
    )(h1r, deg, Wmu)


def _k_kernel(mur, zer, deg, Wf, bf, Wd, bd, orec):
    di = _dinv(deg[...])
    mp = mur[...]
    z = (mp[0] + mp[1]) * di
    zp = zer[...]
    ze = jax.nn.relu(jnp.concatenate([zp[0], zp[1]], axis=1) * di)
    fus = _elu(_mm(jnp.concatenate([z, ze], axis=1), Wf[...]) + bf[...])
    orec[...] = _mm(fus, Wd[...]) + bd[...]


@jax.jit
def _k_call(mur, zer, deg, Wf, bf, Wd, bd):
    return pl.pallas_call(
        _k_kernel,
        grid=(GRID,),
        in_specs=[_split_spec(16), _split_spec(64), _split_spec(16),
                  _full_spec((144, 128)), _full_spec((1, 128)),
                  _full_spec((128, 128)), _full_spec((1, 128))],
        out_specs=_row_spec(128),
        out_shape=jax.ShapeDtypeStruct((N, D), jnp.float32),
    )(mur, zer, deg, Wf, bf, Wd, bd)


def kernel(x_exp, x_img, edge_index, Wg, bg, Wi, bi, Wp, bp, Ws, We, Wig,
           Wv1, Wmu, Wf, bf, Wd, bd):
    degp = _spmm16_call(jnp.ones((NPD, 16), jnp.float32), edge_index)
    sup_sea, sup_seb, sup_sia, sup_sib = _abc_call(
        x_exp, x_img, degp, Wg, bg.reshape(1, -1), Wi, bi.reshape(1, -1), Wp,
        bp.reshape(1, -1), Ws)
    se_raw_a = _spmm_call(sup_sea, edge_index, 64)
    se_raw_b = _spmm_call(sup_seb, edge_index, 64)
    si_raw_a = _spmm_call(sup_sia, edge_index, 64)
    si_raw_b = _spmm_call(sup_sib, edge_index, 64)
    sup_ze, sup_zi = _e_call(se_raw_a, se_raw_b, si_raw_a, si_raw_b, degp,
                             We, Wig)
    ze_raw = _spmm_call(sup_ze, edge_index, 64)
    zi_raw = _spmm16_call(sup_zi, edge_index)
    sup_h1 = _g_call(ze_raw, zi_raw, degp, Wv1)
    h1_raw = _spmm_call(sup_h1, edge_index, 32)
    sup_mu = _i_call(h1_raw, degp, Wmu)
    mu_raw = _spmm16_call(sup_mu, edge_index)
    return _k_call(mu_raw, ze_raw, degp, Wf, bf.reshape(1, -1), Wd,
                   bd.reshape(1, -1))
